# Initial kernel scaffold; baseline (speedup 1.0000x reference)
import jax
import jax.numpy as jnp
from jax.experimental import pallas as pl


def kernel(tokens, table):
    raise NotImplementedError("write your pallas kernel here")


# speedup vs baseline: 7.1604x; 7.1604x over previous
import functools
import math

import jax
import jax.numpy as jnp
from jax import lax
from jax.experimental import pallas as pl
from jax.experimental.pallas import tpu as pltpu
from jax.experimental.pallas import tpu_sc as plsc

VOCAB = 100000
EMB = 128
B = 4096
L = 200
SCALE = math.sqrt(EMB)

NC = 2
NS = 16
NW = NC * NS

BFLAT = B * L
BPW = BFLAT // NW
SUB = 4
CH = SUB * 128
NCHUNK = BPW // CH
B128 = BFLAT // 128


def _scale_table(table):
    def body(t_ref, o_ref):
        o_ref[...] = t_ref[...] * jnp.float32(SCALE)

    return pl.pallas_call(
        body,
        out_shape=jax.ShapeDtypeStruct((VOCAB, EMB), jnp.float32),
        grid=(50,),
        in_specs=[pl.BlockSpec((VOCAB // 50, EMB), lambda i: (i, 0))],
        out_specs=pl.BlockSpec((VOCAB // 50, EMB), lambda i: (i, 0)),
    )(table)


_mesh = plsc.VectorSubcoreMesh(core_axis_name="c", subcore_axis_name="s")


@functools.partial(
    pl.kernel,
    mesh=_mesh,
    out_type=jax.ShapeDtypeStruct((B128, 128, EMB), jnp.float32),
    scratch_types=[
        pltpu.VMEM((SUB, 128), jnp.int32),
        pltpu.VMEM((SUB, 128, EMB), jnp.float32),
        pltpu.SemaphoreType.DMA,
    ],
)
def _gather_sc(tok_hbm, tab_hbm, out_hbm, idx_v, rows_v, sem):
    wid = lax.axis_index("s") * NC + lax.axis_index("c")
    base = wid * (BPW // 128)

    def body(i, carry):
        off = base + i * SUB
        pltpu.sync_copy(tok_hbm.at[pl.ds(off, SUB)], idx_v)
        copies = []
        for j in range(SUB):
            copies.append(
                pltpu.async_copy(tab_hbm.at[idx_v.at[j]], rows_v.at[j], sem)
            )
        for c in copies:
            c.wait()
        pltpu.sync_copy(rows_v, out_hbm.at[pl.ds(off, SUB)])
        return carry

    lax.fori_loop(0, NCHUNK, body, 0)


def kernel(tokens, table):
    scaled = _scale_table(table.astype(jnp.float32))
    tok2d = tokens.reshape(B128, 128).astype(jnp.int32)
    out = _gather_sc(tok2d, scaled)
    return out.reshape(B, L, EMB)


# speedup vs baseline: 7.9488x; 1.1101x over previous
import functools
import math

import jax
import jax.numpy as jnp
from jax import lax
from jax.experimental import pallas as pl
from jax.experimental.pallas import tpu as pltpu
from jax.experimental.pallas import tpu_sc as plsc

VOCAB = 100000
EMB = 128
B = 4096
L = 200
SCALE = math.sqrt(EMB)

NC = 2
NS = 16
NW = NC * NS

BFLAT = B * L
BPW = BFLAT // NW
SUB = 2
CH = SUB * 128
NCHUNK = BPW // CH
IDXROWS = BPW // 128
B128 = BFLAT // 128


def _scale_table(table):
    def body(t_ref, o_ref):
        o_ref[...] = t_ref[...] * jnp.float32(SCALE)

    return pl.pallas_call(
        body,
        out_shape=jax.ShapeDtypeStruct((VOCAB, EMB), jnp.float32),
        grid=(50,),
        in_specs=[pl.BlockSpec((VOCAB // 50, EMB), lambda i: (i, 0))],
        out_specs=pl.BlockSpec((VOCAB // 50, EMB), lambda i: (i, 0)),
    )(table)


_mesh = plsc.VectorSubcoreMesh(core_axis_name="c", subcore_axis_name="s")


@functools.partial(
    pl.kernel,
    mesh=_mesh,
    out_type=jax.ShapeDtypeStruct((B128, 128, EMB), jnp.float32),
    scratch_types=[
        pltpu.VMEM((IDXROWS, 128), jnp.int32),
        pltpu.VMEM((SUB, 128, EMB), jnp.float32),
        pltpu.VMEM((SUB, 128, EMB), jnp.float32),
        pltpu.SemaphoreType.DMA,
        pltpu.SemaphoreType.DMA,
        pltpu.SemaphoreType.DMA,
        pltpu.SemaphoreType.DMA,
    ],
)
def _gather_sc(tok_hbm, tab_hbm, out_hbm, idx_all, rows0, rows1,
               sg0, sg1, so0, so1):
    wid = lax.axis_index("s") * NC + lax.axis_index("c")
    base = wid * (BPW // 128)
    rows = (rows0, rows1)
    sg = (sg0, sg1)
    so = (so0, so1)

    pltpu.sync_copy(tok_hbm.at[pl.ds(base, IDXROWS)], idx_all)

    def fire_gathers(b, c):
        return [
            pltpu.async_copy(
                tab_hbm.at[idx_all.at[c * SUB + j]], rows[b].at[j], sg[b]
            )
            for j in range(SUB)
        ]

    def fire_scatter(b, c):
        pltpu.async_copy(rows[b], out_hbm.at[pl.ds(base + c * SUB, SUB)], so[b])

    def wait_scatter(b, c):
        pltpu.make_async_copy(
            rows[b], out_hbm.at[pl.ds(base + c * SUB, SUB)], so[b]
        ).wait()

    for b in range(2):
        for d in fire_gathers(b, b):
            d.wait()
        fire_scatter(b, b)

    def body(i, carry):
        for b in range(2):
            c = 2 * i + b
            wait_scatter(b, c - 2)
            for d in fire_gathers(b, c):
                d.wait()
            fire_scatter(b, c)
        return carry

    lax.fori_loop(1, NCHUNK // 2, body, 0)

    for b in range(2):
        wait_scatter(b, NCHUNK - 2 + b)


def kernel(tokens, table):
    scaled = _scale_table(table.astype(jnp.float32))
    tok2d = tokens.reshape(B128, 128).astype(jnp.int32)
    out = _gather_sc(tok2d, scaled)
    return out.reshape(B, L, EMB)
